# baseline (speedup 1.0000x reference)
import functools

import jax
import jax.numpy as jnp
from jax import lax
from jax.experimental import pallas as pl
from jax.experimental.pallas import tpu as pltpu
from jax.experimental.pallas import tpu_sc as plsc

N = 10000
EU = 80000
E2 = 2 * EU
DH = 300
DP = 384
HALF = N // 2
ACC = 5120
ZR = ACC // 16


def _mesh():
    return plsc.VectorSubcoreMesh(core_axis_name="c", subcore_axis_name="s")


def _sc_gather(table, idx):
    CH = 128
    NCHG = E2 // CH
    NW = 32
    FULL = NCHG // NW
    EXTRA = NCHG - FULL * NW

    @functools.partial(
        pl.kernel,
        out_type=jax.ShapeDtypeStruct((E2, DP), jnp.float32),
        mesh=_mesh(),
        scratch_types=[
            pltpu.VMEM((CH,), jnp.int32),
            pltpu.VMEM((CH,), jnp.int32),
            pltpu.VMEM((CH, DP), jnp.float32),
            pltpu.VMEM((CH, DP), jnp.float32),
            pltpu.SemaphoreType.DMA,
            pltpu.SemaphoreType.DMA,
            pltpu.SemaphoreType.DMA,
            pltpu.SemaphoreType.DMA,
        ],
    )
    def k(table_hbm, idx_hbm, out_hbm,
          idx_a, idx_b, rows_a, rows_b, sga, sgb, swa, swb):
        w = lax.axis_index("s") * 2 + lax.axis_index("c")

        def gat(chunk, idx_r, rows_r, sem):
            pltpu.sync_copy(idx_hbm.at[pl.ds(chunk * CH, CH)], idx_r)
            pltpu.async_copy(table_hbm.at[idx_r], rows_r, sem)

        def wait_gat(idx_r, rows_r, sem):
            pltpu.make_async_copy(table_hbm.at[idx_r], rows_r, sem).wait()

        def wr(chunk, rows_r, sem):
            pltpu.async_copy(rows_r, out_hbm.at[pl.ds(chunk * CH, CH)], sem)

        def wait_wr(chunk, rows_r, sem):
            pltpu.make_async_copy(
                rows_r, out_hbm.at[pl.ds(chunk * CH, CH)], sem).wait()

        gat(w, idx_a, rows_a, sga)
        gat(w + NW, idx_b, rows_b, sgb)
        wait_gat(idx_a, rows_a, sga)
        wr(w, rows_a, swa)
        wait_gat(idx_b, rows_b, sgb)
        wr(w + NW, rows_b, swb)

        def pair(kk, _):
            ca = w + NW * (2 * kk)
            cb = w + NW * (2 * kk + 1)
            wait_wr(ca - 2 * NW, rows_a, swa)
            gat(ca, idx_a, rows_a, sga)
            wait_gat(idx_a, rows_a, sga)
            wr(ca, rows_a, swa)
            wait_wr(cb - 2 * NW, rows_b, swb)
            gat(cb, idx_b, rows_b, sgb)
            wait_gat(idx_b, rows_b, sgb)
            wr(cb, rows_b, swb)
            return 0

        lax.fori_loop(1, FULL // 2, pair, 0)

        ct_a = w + NW * (FULL - 1)
        wait_wr(ct_a - 2 * NW, rows_a, swa)
        gat(ct_a, idx_a, rows_a, sga)
        wait_gat(idx_a, rows_a, sga)
        wr(ct_a, rows_a, swa)

        @pl.when(w < EXTRA)
        def _():
            ct = w + NW * FULL
            wait_wr(ct - 2 * NW, rows_b, swb)
            gat(ct, idx_b, rows_b, sgb)
            wait_gat(idx_b, rows_b, sgb)
            wr(ct, rows_b, swb)

        wait_wr(0, rows_a, swa)
        wait_wr(0, rows_b, swb)

    return k(table, idx)


def _sc_segsum_partial(rows, idx):
    CH = 64
    NCHC = E2 // CH // 2
    NT = 16
    FULL = NCHC // NT
    EXTRA = NCHC - FULL * NT
    CG = 128
    NBLK = N // 400

    @functools.partial(
        pl.kernel,
        out_type=jax.ShapeDtypeStruct((2 * N, DP), jnp.float32),
        mesh=_mesh(),
        scratch_types=[
            pltpu.VMEM((CH,), jnp.int32),
            pltpu.VMEM((CH,), jnp.int32),
            pltpu.VMEM((CH,), jnp.int32),
            pltpu.VMEM((CH,), jnp.int32),
            pltpu.VMEM((CH, CG), jnp.float32),
            pltpu.VMEM((CH, CG), jnp.float32),
            pltpu.VMEM((CH, CG), jnp.float32),
            pltpu.VMEM((CH, CG), jnp.float32),
            pltpu.VMEM_SHARED((N, CG), jnp.float32),
            pltpu.SemaphoreType.DMA,
            pltpu.SemaphoreType.DMA,
            pltpu.SemaphoreType.DMA,
            pltpu.SemaphoreType.DMA,
            pltpu.SemaphoreType.DMA,
            pltpu.SemaphoreType.DMA,
            pltpu.SemaphoreType.DMA,
            pltpu.SemaphoreType.DMA,
        ],
    )
    def k(rows_hbm, idx_hbm, zeros_hbm, out_hbm,
          i0, i1, i2, i3, r0, r1, r2, r3, acc_sh,
          sl0, sl1, sl2, sl3, ss0, ss1, ss2, ss3):
        c = lax.axis_index("c")
        s = lax.axis_index("s")
        cb0 = c * NCHC
        bufs = [(i0, r0, sl0, ss0), (i1, r1, sl1, ss1),
                (i2, r2, sl2, ss2), (i3, r3, sl3, ss3)]

        def _do_group(g):
            pltpu.sync_copy(zeros_hbm, acc_sh.at[pl.ds(s * 400, 400)])

            @pl.when(s + NT < NBLK)
            def _():
                pltpu.sync_copy(zeros_hbm, acc_sh.at[pl.ds((s + NT) * 400, 400)])

            plsc.subcore_barrier()

            def load(chunk, idx_r, rows_r, sem):
                off = chunk * CH
                pltpu.sync_copy(idx_hbm.at[pl.ds(off, CH)], idx_r)
                pltpu.async_copy(
                    rows_hbm.at[pl.ds(off, CH), pl.ds(g * CG, CG)], rows_r, sem)

            def wait_load(idx_r, rows_r, sem, chunk):
                off = chunk * CH
                pltpu.make_async_copy(
                    rows_hbm.at[pl.ds(off, CH), pl.ds(g * CG, CG)], rows_r, sem
                ).wait()

            def scat(idx_r, rows_r, sem):
                pltpu.async_copy(rows_r, acc_sh.at[idx_r], sem, add=True)

            def wait_scat(idx_r, rows_r, sem):
                pltpu.make_async_copy(rows_r, acc_sh.at[idx_r], sem).wait()

            def fill(k_, bi):
                idx_r, rows_r, sl, ss = bufs[bi]
                ch = cb0 + s + NT * k_
                load(ch, idx_r, rows_r, sl)
                wait_load(idx_r, rows_r, sl, ch)
                scat(idx_r, rows_r, ss)

            def step(k_, bi):
                idx_r, rows_r, sl, ss = bufs[bi]
                ch = cb0 + s + NT * k_
                wait_scat(idx_r, rows_r, ss)
                load(ch, idx_r, rows_r, sl)
                wait_load(idx_r, rows_r, sl, ch)
                scat(idx_r, rows_r, ss)

            for bi in range(4):
                fill(bi, bi)

            def quad(kk, _):
                for bi in range(4):
                    step(4 * kk + bi, bi)
                return 0

            lax.fori_loop(1, FULL // 4, quad, 0)

            step(FULL - 2, 0)
            step(FULL - 1, 1)

            @pl.when(s < EXTRA)
            def _():
                step(FULL, 2)

            for bi in range(4):
                idx_r, rows_r, _sl, ss = bufs[bi]
                wait_scat(idx_r, rows_r, ss)
            plsc.subcore_barrier()

            pltpu.sync_copy(
                acc_sh.at[pl.ds(s * 400, 400)],
                out_hbm.at[pl.ds(c * N + s * 400, 400), pl.ds(g * CG, CG)])

            @pl.when(s + NT < NBLK)
            def _():
                pltpu.sync_copy(
                    acc_sh.at[pl.ds((s + NT) * 400, 400)],
                    out_hbm.at[pl.ds(c * N + (s + NT) * 400, 400),
                               pl.ds(g * CG, CG)])

            plsc.subcore_barrier()

        for g in range(DP // CG):
            _do_group(g)

    zeros = jnp.zeros((400, CG), jnp.float32)
    return k(rows, idx, zeros)


def _sc_combine_gather(partials, idx):
    CH = 64
    NCHC = E2 // CH // 2
    NT = 16
    FULL = NCHC // NT
    EXTRA = NCHC - FULL * NT
    CG = 128
    NBLK = N // 400
    NCB = N // CH
    TAIL = N - NCB * CH

    @functools.partial(
        pl.kernel,
        out_type=jax.ShapeDtypeStruct((E2, DP), jnp.float32),
        mesh=_mesh(),
        scratch_types=[
            pltpu.VMEM((CH,), jnp.int32),
            pltpu.VMEM((CH,), jnp.int32),
            pltpu.VMEM((CH,), jnp.int32),
            pltpu.VMEM((CH,), jnp.int32),
            pltpu.VMEM((CH, CG), jnp.float32),
            pltpu.VMEM((CH, CG), jnp.float32),
            pltpu.VMEM((CH, CG), jnp.float32),
            pltpu.VMEM((CH, CG), jnp.float32),
            pltpu.VMEM_SHARED((N, CG), jnp.float32),
            pltpu.SemaphoreType.DMA,
            pltpu.SemaphoreType.DMA,
            pltpu.SemaphoreType.DMA,
            pltpu.SemaphoreType.DMA,
            pltpu.SemaphoreType.DMA,
            pltpu.SemaphoreType.DMA,
            pltpu.SemaphoreType.DMA,
            pltpu.SemaphoreType.DMA,
        ],
    )
    def k(part_hbm, idx_hbm, iota_hbm, out_hbm,
          i0, i1, i2, i3, r0, r1, r2, r3, acc_sh,
          sl0, sl1, sl2, sl3, ss0, ss1, ss2, ss3):
        c = lax.axis_index("c")
        s = lax.axis_index("s")
        cb0 = c * NCHC
        bufs = [(i0, r0, sl0, ss0), (i1, r1, sl1, ss1),
                (i2, r2, sl2, ss2), (i3, r3, sl3, ss3)]
        idx_a, rows_a, sla, ssa = bufs[0]

        def _do_group(g):
            pltpu.sync_copy(
                part_hbm.at[pl.ds(s * 400, 400), pl.ds(g * CG, CG)],
                acc_sh.at[pl.ds(s * 400, 400)])

            @pl.when(s + NT < NBLK)
            def _():
                pltpu.sync_copy(
                    part_hbm.at[pl.ds((s + NT) * 400, 400), pl.ds(g * CG, CG)],
                    acc_sh.at[pl.ds((s + NT) * 400, 400)])

            plsc.subcore_barrier()

            def add_chunk(chk, ln):
                offs = idx_a.at[pl.ds(0, ln)]
                pltpu.sync_copy(iota_hbm.at[pl.ds(chk * CH, ln)], offs)
                pltpu.sync_copy(
                    part_hbm.at[pl.ds(N + chk * CH, ln), pl.ds(g * CG, CG)],
                    rows_a.at[pl.ds(0, ln)])
                pltpu.async_copy(
                    rows_a.at[pl.ds(0, ln)], acc_sh.at[offs], ssa, add=True)
                pltpu.make_async_copy(
                    rows_a.at[pl.ds(0, ln)], acc_sh.at[offs], ssa).wait()

            for j in range(NCB // NT):
                add_chunk(s + NT * j, CH)

            @pl.when(s < NCB - (NCB // NT) * NT)
            def _():
                add_chunk((NCB // NT) * NT + s, CH)

            @pl.when(s == NT - 1)
            def _():
                add_chunk(NCB, TAIL)

            plsc.subcore_barrier()

            def gat(chunk, idx_r, rows_r, sem):
                pltpu.sync_copy(idx_hbm.at[pl.ds(chunk * CH, CH)], idx_r)
                pltpu.async_copy(acc_sh.at[idx_r], rows_r, sem)

            def wait_gat(idx_r, rows_r, sem):
                pltpu.make_async_copy(acc_sh.at[idx_r], rows_r, sem).wait()

            def wr(chunk, rows_r, sem):
                pltpu.async_copy(
                    rows_r,
                    out_hbm.at[pl.ds(chunk * CH, CH), pl.ds(g * CG, CG)], sem)

            def wait_wr(chunk, rows_r, sem):
                pltpu.make_async_copy(
                    rows_r,
                    out_hbm.at[pl.ds(chunk * CH, CH), pl.ds(g * CG, CG)],
                    sem).wait()

            def fill_g(k_, bi):
                idx_r, rows_r, sl, ss = bufs[bi]
                ch = cb0 + s + NT * k_
                gat(ch, idx_r, rows_r, sl)
                wait_gat(idx_r, rows_r, sl)
                wr(ch, rows_r, ss)

            def step_g(k_, bi):
                idx_r, rows_r, sl, ss = bufs[bi]
                ch = cb0 + s + NT * k_
                wait_wr(ch - 4 * NT, rows_r, ss)
                gat(ch, idx_r, rows_r, sl)
                wait_gat(idx_r, rows_r, sl)
                wr(ch, rows_r, ss)

            for bi in range(4):
                fill_g(bi, bi)

            def quad(kk, _):
                for bi in range(4):
                    step_g(4 * kk + bi, bi)
                return 0

            lax.fori_loop(1, FULL // 4, quad, 0)

            step_g(FULL - 2, 0)
            step_g(FULL - 1, 1)

            @pl.when(s < EXTRA)
            def _():
                step_g(FULL, 2)

            wait_wr(cb0 + s + NT * (FULL - 2), bufs[0][1], bufs[0][3])
            wait_wr(cb0 + s + NT * (FULL - 1), bufs[1][1], bufs[1][3])
            wait_wr(cb0 + s + NT * (FULL - 3), bufs[3][1], bufs[3][3])

            @pl.when(s < EXTRA)
            def _():
                wait_wr(cb0 + s + NT * FULL, bufs[2][1], bufs[2][3])

            @pl.when(s >= EXTRA)
            def _():
                wait_wr(cb0 + s + NT * (FULL - 4), bufs[2][1], bufs[2][3])

            plsc.subcore_barrier()

        for g in range(DP // CG):
            _do_group(g)

    return k(partials, idx, jnp.arange(N, dtype=jnp.int32))


def _mm_small(x, w):
    B = 1000
    K = x.shape[1]
    W = w.shape[1]

    def body(x_ref, w_ref, o_ref):
        o_ref[...] = jnp.dot(x_ref[...], w_ref[...],
                             preferred_element_type=jnp.float32)

    return pl.pallas_call(
        body,
        grid=(x.shape[0] // B,),
        in_specs=[pl.BlockSpec((B, K), lambda i: (i, 0)),
                  pl.BlockSpec((K, W), lambda i: (0, 0))],
        out_specs=pl.BlockSpec((B, W), lambda i: (i, 0)),
        out_shape=jax.ShapeDtypeStruct((x.shape[0], W), jnp.float32),
    )(x, w)


def _tc_init(pg, e2, w_ie):
    B = 1000
    DE = e2.shape[1]

    def body(pg_ref, e_ref, w_ref, h0_ref, h_ref):
        h0 = pg_ref[...] + jnp.dot(e_ref[...], w_ref[...],
                                   preferred_element_type=jnp.float32)
        h0_ref[...] = h0
        h_ref[...] = jnp.maximum(h0, 0.0)

    return pl.pallas_call(
        body,
        grid=(E2 // B,),
        in_specs=[pl.BlockSpec((B, DP), lambda i: (i, 0)),
                  pl.BlockSpec((B, DE), lambda i: (i, 0)),
                  pl.BlockSpec((DE, DP), lambda i: (0, 0))],
        out_specs=[pl.BlockSpec((B, DP), lambda i: (i, 0)),
                   pl.BlockSpec((B, DP), lambda i: (i, 0))],
        out_shape=[jax.ShapeDtypeStruct((E2, DP), jnp.float32),
                   jax.ShapeDtypeStruct((E2, DP), jnp.float32)],
    )(pg, e2, w_ie)


def _tc_step(mg, h, h0, w_h):
    B = 1000
    NB = E2 // B

    def body(mg_ref, hsw_ref, h0_ref, w_ref, o_ref):
        x = mg_ref[...] - hsw_ref[...]
        y = jnp.dot(x, w_ref[...], preferred_element_type=jnp.float32)
        o_ref[...] = jnp.maximum(h0_ref[...] + y, 0.0)

    return pl.pallas_call(
        body,
        grid=(NB,),
        in_specs=[pl.BlockSpec((B, DP), lambda i: (i, 0)),
                  pl.BlockSpec((B, DP), lambda i: ((i + NB // 2) % NB, 0)),
                  pl.BlockSpec((B, DP), lambda i: (i, 0)),
                  pl.BlockSpec((DP, DP), lambda i: (0, 0))],
        out_specs=pl.BlockSpec((B, DP), lambda i: (i, 0)),
        out_shape=jax.ShapeDtypeStruct((E2, DP), jnp.float32),
    )(mg, h, h0, w_h)


def _tc_final(v, parts, w_ov, w_om, b_o):
    B = 1000
    DV = v.shape[1]
    NB = N // B

    def body(v_ref, m0_ref, m1_ref, wv_ref, wm_ref, b_ref, o_ref):
        y = jnp.dot(v_ref[...], wv_ref[...], preferred_element_type=jnp.float32)
        mv = m0_ref[...] + m1_ref[...]
        y += jnp.dot(mv, wm_ref[...], preferred_element_type=jnp.float32)
        o_ref[...] = jnp.maximum(y + b_ref[...], 0.0)

    return pl.pallas_call(
        body,
        grid=(NB,),
        in_specs=[pl.BlockSpec((B, DV), lambda i: (i, 0)),
                  pl.BlockSpec((B, DP), lambda i: (i, 0)),
                  pl.BlockSpec((B, DP), lambda i: (i + NB, 0)),
                  pl.BlockSpec((DV, DH), lambda i: (0, 0)),
                  pl.BlockSpec((DP, DH), lambda i: (0, 0)),
                  pl.BlockSpec((1, DH), lambda i: (0, 0))],
        out_specs=pl.BlockSpec((B, DH), lambda i: (i, 0)),
        out_shape=jax.ShapeDtypeStruct((N, DH), jnp.float32),
    )(v, parts, parts, w_ov, w_om, b_o)


def kernel(V, E, edge_index, rev_edge_index, W_i, W_h, W_o, b_o):
    del rev_edge_index
    DV = V.shape[1]
    src = edge_index[0]
    dst = edge_index[1]
    src2 = src.reshape(-1, 2).T.reshape(-1)
    dst2 = dst.reshape(-1, 2).T.reshape(-1)
    e2 = E.reshape(-1, 2, E.shape[1]).transpose(1, 0, 2).reshape(E2, -1)

    pad = ((0, 0), (0, DP - DH))
    w_iv = jnp.pad(W_i[:DV], pad)
    w_ie = jnp.pad(W_i[DV:], pad)
    w_h = jnp.pad(W_h, ((0, DP - DH), (0, DP - DH)))
    w_ov = W_o[:DV]
    w_om = jnp.pad(W_o[DV:], ((0, DP - DH), (0, 0)))

    p = _mm_small(V, w_iv)
    pg = _sc_gather(p, src2)
    h0, h = _tc_init(pg, e2, w_ie)

    for _ in range(2):
        part = _sc_segsum_partial(h, dst2)
        mg = _sc_combine_gather(part, src2)
        h = _tc_step(mg, h, h0, w_h)

    part = _sc_segsum_partial(h, dst2)
    return _tc_final(V, part, w_ov, w_om, b_o.reshape(1, DH))


# speedup vs baseline: 1.1963x; 1.1963x over previous
import functools

import jax
import jax.numpy as jnp
from jax import lax
from jax.experimental import pallas as pl
from jax.experimental.pallas import tpu as pltpu
from jax.experimental.pallas import tpu_sc as plsc

N = 10000
EU = 80000
E2 = 2 * EU
DH = 300
DP = 384
HALF = N // 2
ACC = 5120
ZR = ACC // 16


def _mesh():
    return plsc.VectorSubcoreMesh(core_axis_name="c", subcore_axis_name="s")


def _sc_gather(table, idx):
    CH = 128
    NCHG = E2 // CH
    NW = 32
    FULL = NCHG // NW
    EXTRA = NCHG - FULL * NW

    @functools.partial(
        pl.kernel,
        out_type=jax.ShapeDtypeStruct((E2, DP), jnp.float32),
        mesh=_mesh(),
        scratch_types=[
            pltpu.VMEM((CH,), jnp.int32),
            pltpu.VMEM((CH,), jnp.int32),
            pltpu.VMEM((CH, DP), jnp.float32),
            pltpu.VMEM((CH, DP), jnp.float32),
            pltpu.SemaphoreType.DMA,
            pltpu.SemaphoreType.DMA,
            pltpu.SemaphoreType.DMA,
            pltpu.SemaphoreType.DMA,
        ],
    )
    def k(table_hbm, idx_hbm, out_hbm,
          idx_a, idx_b, rows_a, rows_b, sga, sgb, swa, swb):
        w = lax.axis_index("s") * 2 + lax.axis_index("c")

        def gat(chunk, idx_r, rows_r, sem):
            pltpu.sync_copy(idx_hbm.at[pl.ds(chunk * CH, CH)], idx_r)
            pltpu.async_copy(table_hbm.at[idx_r], rows_r, sem)

        def wait_gat(idx_r, rows_r, sem):
            pltpu.make_async_copy(table_hbm.at[idx_r], rows_r, sem).wait()

        def wr(chunk, rows_r, sem):
            pltpu.async_copy(rows_r, out_hbm.at[pl.ds(chunk * CH, CH)], sem)

        def wait_wr(chunk, rows_r, sem):
            pltpu.make_async_copy(
                rows_r, out_hbm.at[pl.ds(chunk * CH, CH)], sem).wait()

        gat(w, idx_a, rows_a, sga)
        gat(w + NW, idx_b, rows_b, sgb)
        wait_gat(idx_a, rows_a, sga)
        wr(w, rows_a, swa)
        wait_gat(idx_b, rows_b, sgb)
        wr(w + NW, rows_b, swb)

        def pair(kk, _):
            ca = w + NW * (2 * kk)
            cb = w + NW * (2 * kk + 1)
            wait_wr(ca - 2 * NW, rows_a, swa)
            gat(ca, idx_a, rows_a, sga)
            wait_gat(idx_a, rows_a, sga)
            wr(ca, rows_a, swa)
            wait_wr(cb - 2 * NW, rows_b, swb)
            gat(cb, idx_b, rows_b, sgb)
            wait_gat(idx_b, rows_b, sgb)
            wr(cb, rows_b, swb)
            return 0

        lax.fori_loop(1, FULL // 2, pair, 0)

        ct_a = w + NW * (FULL - 1)
        wait_wr(ct_a - 2 * NW, rows_a, swa)
        gat(ct_a, idx_a, rows_a, sga)
        wait_gat(idx_a, rows_a, sga)
        wr(ct_a, rows_a, swa)

        @pl.when(w < EXTRA)
        def _():
            ct = w + NW * FULL
            wait_wr(ct - 2 * NW, rows_b, swb)
            gat(ct, idx_b, rows_b, sgb)
            wait_gat(idx_b, rows_b, sgb)
            wr(ct, rows_b, swb)

        wait_wr(0, rows_a, swa)
        wait_wr(0, rows_b, swb)

    return k(table, idx)


def _sc_segsum_partial(rows, idx):
    CH = 128
    NCHC = E2 // CH // 2
    NT = 16
    FULL = NCHC // NT
    EXTRA = NCHC - FULL * NT
    CG = 128
    NBLK = N // 400

    @functools.partial(
        pl.kernel,
        out_type=jax.ShapeDtypeStruct((2 * N, DP), jnp.float32),
        mesh=_mesh(),
        scratch_types=[
            pltpu.VMEM((CH,), jnp.int32),
            pltpu.VMEM((CH,), jnp.int32),
            pltpu.VMEM((CH, CG), jnp.float32),
            pltpu.VMEM((CH, CG), jnp.float32),
            pltpu.VMEM_SHARED((N, CG), jnp.float32),
            pltpu.SemaphoreType.DMA,
            pltpu.SemaphoreType.DMA,
            pltpu.SemaphoreType.DMA,
            pltpu.SemaphoreType.DMA,
            pltpu.SemaphoreType.DMA,
            pltpu.SemaphoreType.DMA,
        ],
    )
    def k(rows_hbm, idx_hbm, zeros_hbm, out_hbm,
          idx_a, idx_b, rows_a, rows_b, acc_sh,
          sia, sib, sla, slb, ssa, ssb):
        c = lax.axis_index("c")
        s = lax.axis_index("s")
        cb0 = c * NCHC

        def _do_group(g):
            pltpu.sync_copy(zeros_hbm, acc_sh.at[pl.ds(s * 400, 400)])

            @pl.when(s + NT < NBLK)
            def _():
                pltpu.sync_copy(zeros_hbm, acc_sh.at[pl.ds((s + NT) * 400, 400)])

            plsc.subcore_barrier()

            def loads(chunk, idx_r, rows_r, si, sl):
                off = chunk * CH
                pltpu.async_copy(idx_hbm.at[pl.ds(off, CH)], idx_r, si)
                pltpu.async_copy(
                    rows_hbm.at[pl.ds(off, CH), pl.ds(g * CG, CG)], rows_r, sl)

            def wait_loads(idx_r, rows_r, si, sl, chunk):
                off = chunk * CH
                pltpu.make_async_copy(
                    idx_hbm.at[pl.ds(off, CH)], idx_r, si).wait()
                pltpu.make_async_copy(
                    rows_hbm.at[pl.ds(off, CH), pl.ds(g * CG, CG)], rows_r, sl
                ).wait()

            def scat(idx_r, rows_r, sem):
                pltpu.async_copy(rows_r, acc_sh.at[idx_r], sem, add=True)

            def wait_scat(idx_r, rows_r, sem):
                pltpu.make_async_copy(rows_r, acc_sh.at[idx_r], sem).wait()

            loads(cb0 + s, idx_a, rows_a, sia, sla)
            loads(cb0 + s + NT, idx_b, rows_b, sib, slb)
            wait_loads(idx_a, rows_a, sia, sla, cb0 + s)
            scat(idx_a, rows_a, ssa)
            wait_loads(idx_b, rows_b, sib, slb, cb0 + s + NT)
            scat(idx_b, rows_b, ssb)

            def duo(kk, _):
                ca = cb0 + s + NT * (2 * kk)
                cb = cb0 + s + NT * (2 * kk + 1)
                wait_scat(idx_a, rows_a, ssa)
                loads(ca, idx_a, rows_a, sia, sla)
                wait_scat(idx_b, rows_b, ssb)
                loads(cb, idx_b, rows_b, sib, slb)
                wait_loads(idx_a, rows_a, sia, sla, ca)
                scat(idx_a, rows_a, ssa)
                wait_loads(idx_b, rows_b, sib, slb, cb)
                scat(idx_b, rows_b, ssb)
                return 0

            lax.fori_loop(1, FULL // 2, duo, 0)

            ct_a = cb0 + s + NT * (FULL - 1)
            wait_scat(idx_a, rows_a, ssa)
            loads(ct_a, idx_a, rows_a, sia, sla)
            wait_loads(idx_a, rows_a, sia, sla, ct_a)
            scat(idx_a, rows_a, ssa)

            @pl.when(s < EXTRA)
            def _():
                ct = cb0 + s + NT * FULL
                wait_scat(idx_b, rows_b, ssb)
                loads(ct, idx_b, rows_b, sib, slb)
                wait_loads(idx_b, rows_b, sib, slb, ct)
                scat(idx_b, rows_b, ssb)

            wait_scat(idx_a, rows_a, ssa)
            wait_scat(idx_b, rows_b, ssb)
            plsc.subcore_barrier()

            pltpu.sync_copy(
                acc_sh.at[pl.ds(s * 400, 400)],
                out_hbm.at[pl.ds(c * N + s * 400, 400), pl.ds(g * CG, CG)])

            @pl.when(s + NT < NBLK)
            def _():
                pltpu.sync_copy(
                    acc_sh.at[pl.ds((s + NT) * 400, 400)],
                    out_hbm.at[pl.ds(c * N + (s + NT) * 400, 400),
                               pl.ds(g * CG, CG)])

            plsc.subcore_barrier()

        for g in range(DP // CG):
            _do_group(g)

    zeros = jnp.zeros((400, CG), jnp.float32)
    return k(rows, idx, zeros)


def _sc_combine_gather(partials, idx):
    CH = 128
    NCHC = E2 // CH // 2
    NT = 16
    FULL = NCHC // NT
    EXTRA = NCHC - FULL * NT
    CG = 128
    NBLK = N // 400
    NCB = N // CH
    TAIL = N - NCB * CH

    @functools.partial(
        pl.kernel,
        out_type=jax.ShapeDtypeStruct((E2, DP), jnp.float32),
        mesh=_mesh(),
        scratch_types=[
            pltpu.VMEM((CH,), jnp.int32),
            pltpu.VMEM((CH,), jnp.int32),
            pltpu.VMEM((CH, CG), jnp.float32),
            pltpu.VMEM((CH, CG), jnp.float32),
            pltpu.VMEM_SHARED((N, CG), jnp.float32),
            pltpu.SemaphoreType.DMA,
            pltpu.SemaphoreType.DMA,
            pltpu.SemaphoreType.DMA,
            pltpu.SemaphoreType.DMA,
            pltpu.SemaphoreType.DMA,
            pltpu.SemaphoreType.DMA,
        ],
    )
    def k(part_hbm, idx_hbm, iota_hbm, out_hbm,
          idx_a, idx_b, rows_a, rows_b, acc_sh,
          sia, sib, sla, slb, ssa, ssb):
        c = lax.axis_index("c")
        s = lax.axis_index("s")
        cb0 = c * NCHC

        def _do_group(g):
            pltpu.sync_copy(
                part_hbm.at[pl.ds(s * 400, 400), pl.ds(g * CG, CG)],
                acc_sh.at[pl.ds(s * 400, 400)])

            @pl.when(s + NT < NBLK)
            def _():
                pltpu.sync_copy(
                    part_hbm.at[pl.ds((s + NT) * 400, 400), pl.ds(g * CG, CG)],
                    acc_sh.at[pl.ds((s + NT) * 400, 400)])

            plsc.subcore_barrier()

            def add_chunk(chk, ln):
                offs = idx_a.at[pl.ds(0, ln)]
                pltpu.sync_copy(iota_hbm.at[pl.ds(chk * CH, ln)], offs)
                pltpu.sync_copy(
                    part_hbm.at[pl.ds(N + chk * CH, ln), pl.ds(g * CG, CG)],
                    rows_a.at[pl.ds(0, ln)])
                pltpu.async_copy(
                    rows_a.at[pl.ds(0, ln)], acc_sh.at[offs], ssa, add=True)
                pltpu.make_async_copy(
                    rows_a.at[pl.ds(0, ln)], acc_sh.at[offs], ssa).wait()

            for j in range(NCB // NT):
                add_chunk(s + NT * j, CH)

            @pl.when(s < NCB - (NCB // NT) * NT)
            def _():
                add_chunk((NCB // NT) * NT + s, CH)

            @pl.when(s == NT - 1)
            def _():
                add_chunk(NCB, TAIL)

            plsc.subcore_barrier()

            def gat(chunk, idx_r, rows_r, sem):
                del chunk
                pltpu.async_copy(acc_sh.at[idx_r], rows_r, sem)

            def wait_gat(idx_r, rows_r, sem):
                pltpu.make_async_copy(acc_sh.at[idx_r], rows_r, sem).wait()

            def wr(chunk, rows_r, sem):
                pltpu.async_copy(
                    rows_r,
                    out_hbm.at[pl.ds(chunk * CH, CH), pl.ds(g * CG, CG)], sem)

            def wait_wr(chunk, rows_r, sem):
                pltpu.make_async_copy(
                    rows_r,
                    out_hbm.at[pl.ds(chunk * CH, CH), pl.ds(g * CG, CG)],
                    sem).wait()

            def aidx(chunk, idx_r, si):
                pltpu.async_copy(idx_hbm.at[pl.ds(chunk * CH, CH)], idx_r, si)

            def wait_aidx(chunk, idx_r, si):
                pltpu.make_async_copy(
                    idx_hbm.at[pl.ds(chunk * CH, CH)], idx_r, si).wait()

            aidx(cb0 + s, idx_a, sia)
            aidx(cb0 + s + NT, idx_b, sib)
            wait_aidx(cb0 + s, idx_a, sia)
            gat(cb0 + s, idx_a, rows_a, sla)
            wait_aidx(cb0 + s + NT, idx_b, sib)
            gat(cb0 + s + NT, idx_b, rows_b, slb)
            wait_gat(idx_a, rows_a, sla)
            wr(cb0 + s, rows_a, ssa)
            wait_gat(idx_b, rows_b, slb)
            wr(cb0 + s + NT, rows_b, ssb)

            def duo(kk, _):
                ca = cb0 + s + NT * (2 * kk)
                cb = cb0 + s + NT * (2 * kk + 1)
                wait_wr(ca - 2 * NT, rows_a, ssa)
                aidx(ca, idx_a, sia)
                wait_wr(cb - 2 * NT, rows_b, ssb)
                aidx(cb, idx_b, sib)
                wait_aidx(ca, idx_a, sia)
                gat(ca, idx_a, rows_a, sla)
                wait_aidx(cb, idx_b, sib)
                gat(cb, idx_b, rows_b, slb)
                wait_gat(idx_a, rows_a, sla)
                wr(ca, rows_a, ssa)
                wait_gat(idx_b, rows_b, slb)
                wr(cb, rows_b, ssb)
                return 0

            lax.fori_loop(1, FULL // 2, duo, 0)

            ct_a = cb0 + s + NT * (FULL - 1)
            wait_wr(ct_a - 2 * NT, rows_a, ssa)
            aidx(ct_a, idx_a, sia)
            wait_aidx(ct_a, idx_a, sia)
            gat(ct_a, idx_a, rows_a, sla)
            wait_gat(idx_a, rows_a, sla)
            wr(ct_a, rows_a, ssa)

            @pl.when(s < EXTRA)
            def _():
                ct = cb0 + s + NT * FULL
                wait_wr(ct - 2 * NT, rows_b, ssb)
                aidx(ct, idx_b, sib)
                wait_aidx(ct, idx_b, sib)
                gat(ct, idx_b, rows_b, slb)
                wait_gat(idx_b, rows_b, slb)
                wr(ct, rows_b, ssb)

            wait_wr(ct_a, rows_a, ssa)

            @pl.when(s < EXTRA)
            def _():
                wait_wr(cb0 + s + NT * FULL, rows_b, ssb)

            @pl.when(s >= EXTRA)
            def _():
                wait_wr(cb0 + s + NT * (FULL - 2), rows_b, ssb)

            plsc.subcore_barrier()

        for g in range(DP // CG):
            _do_group(g)

    return k(partials, idx, jnp.arange(N, dtype=jnp.int32))


def _mm_small(x, w):
    B = 1000
    K = x.shape[1]
    W = w.shape[1]

    def body(x_ref, w_ref, o_ref):
        o_ref[...] = jnp.dot(x_ref[...], w_ref[...],
                             preferred_element_type=jnp.float32)

    return pl.pallas_call(
        body,
        grid=(x.shape[0] // B,),
        in_specs=[pl.BlockSpec((B, K), lambda i: (i, 0)),
                  pl.BlockSpec((K, W), lambda i: (0, 0))],
        out_specs=pl.BlockSpec((B, W), lambda i: (i, 0)),
        out_shape=jax.ShapeDtypeStruct((x.shape[0], W), jnp.float32),
    )(x, w)


def _tc_init(pg, e2, w_ie):
    B = 1000
    DE = e2.shape[1]

    def body(pg_ref, e_ref, w_ref, h0_ref, h_ref):
        h0 = pg_ref[...] + jnp.dot(e_ref[...], w_ref[...],
                                   preferred_element_type=jnp.float32)
        h0_ref[...] = h0
        h_ref[...] = jnp.maximum(h0, 0.0)

    return pl.pallas_call(
        body,
        grid=(E2 // B,),
        in_specs=[pl.BlockSpec((B, DP), lambda i: (i, 0)),
                  pl.BlockSpec((B, DE), lambda i: (i, 0)),
                  pl.BlockSpec((DE, DP), lambda i: (0, 0))],
        out_specs=[pl.BlockSpec((B, DP), lambda i: (i, 0)),
                   pl.BlockSpec((B, DP), lambda i: (i, 0))],
        out_shape=[jax.ShapeDtypeStruct((E2, DP), jnp.float32),
                   jax.ShapeDtypeStruct((E2, DP), jnp.float32)],
    )(pg, e2, w_ie)


def _tc_step(mg, h, h0, w_h):
    B = 1000
    NB = E2 // B

    def body(mg_ref, hsw_ref, h0_ref, w_ref, o_ref):
        x = mg_ref[...] - hsw_ref[...]
        y = jnp.dot(x, w_ref[...], preferred_element_type=jnp.float32)
        o_ref[...] = jnp.maximum(h0_ref[...] + y, 0.0)

    return pl.pallas_call(
        body,
        grid=(NB,),
        in_specs=[pl.BlockSpec((B, DP), lambda i: (i, 0)),
                  pl.BlockSpec((B, DP), lambda i: ((i + NB // 2) % NB, 0)),
                  pl.BlockSpec((B, DP), lambda i: (i, 0)),
                  pl.BlockSpec((DP, DP), lambda i: (0, 0))],
        out_specs=pl.BlockSpec((B, DP), lambda i: (i, 0)),
        out_shape=jax.ShapeDtypeStruct((E2, DP), jnp.float32),
    )(mg, h, h0, w_h)


def _tc_final(v, parts, w_ov, w_om, b_o):
    B = 1000
    DV = v.shape[1]
    NB = N // B

    def body(v_ref, m0_ref, m1_ref, wv_ref, wm_ref, b_ref, o_ref):
        y = jnp.dot(v_ref[...], wv_ref[...], preferred_element_type=jnp.float32)
        mv = m0_ref[...] + m1_ref[...]
        y += jnp.dot(mv, wm_ref[...], preferred_element_type=jnp.float32)
        o_ref[...] = jnp.maximum(y + b_ref[...], 0.0)

    return pl.pallas_call(
        body,
        grid=(NB,),
        in_specs=[pl.BlockSpec((B, DV), lambda i: (i, 0)),
                  pl.BlockSpec((B, DP), lambda i: (i, 0)),
                  pl.BlockSpec((B, DP), lambda i: (i + NB, 0)),
                  pl.BlockSpec((DV, DH), lambda i: (0, 0)),
                  pl.BlockSpec((DP, DH), lambda i: (0, 0)),
                  pl.BlockSpec((1, DH), lambda i: (0, 0))],
        out_specs=pl.BlockSpec((B, DH), lambda i: (i, 0)),
        out_shape=jax.ShapeDtypeStruct((N, DH), jnp.float32),
    )(v, parts, parts, w_ov, w_om, b_o)


def kernel(V, E, edge_index, rev_edge_index, W_i, W_h, W_o, b_o):
    del rev_edge_index
    DV = V.shape[1]
    src = edge_index[0]
    dst = edge_index[1]
    src2 = src.reshape(-1, 2).T.reshape(-1)
    dst2 = dst.reshape(-1, 2).T.reshape(-1)
    e2 = E.reshape(-1, 2, E.shape[1]).transpose(1, 0, 2).reshape(E2, -1)

    pad = ((0, 0), (0, DP - DH))
    w_iv = jnp.pad(W_i[:DV], pad)
    w_ie = jnp.pad(W_i[DV:], pad)
    w_h = jnp.pad(W_h, ((0, DP - DH), (0, DP - DH)))
    w_ov = W_o[:DV]
    w_om = jnp.pad(W_o[DV:], ((0, DP - DH), (0, 0)))

    p = _mm_small(V, w_iv)
    pg = _sc_gather(p, src2)
    h0, h = _tc_init(pg, e2, w_ie)

    for _ in range(2):
        part = _sc_segsum_partial(h, dst2)
        mg = _sc_combine_gather(part, src2)
        h = _tc_step(mg, h, h0, w_h)

    part = _sc_segsum_partial(h, dst2)
    return _tc_final(V, part, w_ov, w_om, b_o.reshape(1, DH))


# speedup vs baseline: 1.2326x; 1.0303x over previous
import functools

import jax
import jax.numpy as jnp
from jax import lax
from jax.experimental import pallas as pl
from jax.experimental.pallas import tpu as pltpu
from jax.experimental.pallas import tpu_sc as plsc

N = 10000
EU = 80000
E2 = 2 * EU
DH = 300
DP = 384
HALF = N // 2
ACC = 5120
ZR = ACC // 16


def _mesh():
    return plsc.VectorSubcoreMesh(core_axis_name="c", subcore_axis_name="s")


def _sc_gather(table, idx):
    CH = 128
    NCHG = E2 // CH
    NW = 32
    FULL = NCHG // NW
    EXTRA = NCHG - FULL * NW

    @functools.partial(
        pl.kernel,
        out_type=jax.ShapeDtypeStruct((E2, DP), jnp.float32),
        mesh=_mesh(),
        scratch_types=[
            pltpu.VMEM((CH,), jnp.int32),
            pltpu.VMEM((CH,), jnp.int32),
            pltpu.VMEM((CH, DP), jnp.float32),
            pltpu.VMEM((CH, DP), jnp.float32),
            pltpu.SemaphoreType.DMA,
            pltpu.SemaphoreType.DMA,
            pltpu.SemaphoreType.DMA,
            pltpu.SemaphoreType.DMA,
        ],
    )
    def k(table_hbm, idx_hbm, out_hbm,
          idx_a, idx_b, rows_a, rows_b, sga, sgb, swa, swb):
        w = lax.axis_index("s") * 2 + lax.axis_index("c")

        def gat(chunk, idx_r, rows_r, sem):
            pltpu.sync_copy(idx_hbm.at[pl.ds(chunk * CH, CH)], idx_r)
            pltpu.async_copy(table_hbm.at[idx_r], rows_r, sem)

        def wait_gat(idx_r, rows_r, sem):
            pltpu.make_async_copy(table_hbm.at[idx_r], rows_r, sem).wait()

        def wr(chunk, rows_r, sem):
            pltpu.async_copy(rows_r, out_hbm.at[pl.ds(chunk * CH, CH)], sem)

        def wait_wr(chunk, rows_r, sem):
            pltpu.make_async_copy(
                rows_r, out_hbm.at[pl.ds(chunk * CH, CH)], sem).wait()

        gat(w, idx_a, rows_a, sga)
        gat(w + NW, idx_b, rows_b, sgb)
        wait_gat(idx_a, rows_a, sga)
        wr(w, rows_a, swa)
        wait_gat(idx_b, rows_b, sgb)
        wr(w + NW, rows_b, swb)

        def pair(kk, _):
            ca = w + NW * (2 * kk)
            cb = w + NW * (2 * kk + 1)
            wait_wr(ca - 2 * NW, rows_a, swa)
            gat(ca, idx_a, rows_a, sga)
            wait_gat(idx_a, rows_a, sga)
            wr(ca, rows_a, swa)
            wait_wr(cb - 2 * NW, rows_b, swb)
            gat(cb, idx_b, rows_b, sgb)
            wait_gat(idx_b, rows_b, sgb)
            wr(cb, rows_b, swb)
            return 0

        lax.fori_loop(1, FULL // 2, pair, 0)

        ct_a = w + NW * (FULL - 1)
        wait_wr(ct_a - 2 * NW, rows_a, swa)
        gat(ct_a, idx_a, rows_a, sga)
        wait_gat(idx_a, rows_a, sga)
        wr(ct_a, rows_a, swa)

        @pl.when(w < EXTRA)
        def _():
            ct = w + NW * FULL
            wait_wr(ct - 2 * NW, rows_b, swb)
            gat(ct, idx_b, rows_b, sgb)
            wait_gat(idx_b, rows_b, sgb)
            wr(ct, rows_b, swb)

        wait_wr(0, rows_a, swa)
        wait_wr(0, rows_b, swb)

    return k(table, idx)


def _sc_segsum_partial(rows, idx):
    CH = 128
    NCHC = E2 // CH // 2
    NT = 16
    FULL = NCHC // NT
    EXTRA = NCHC - FULL * NT
    CG = 128
    NBLK = N // 400

    @functools.partial(
        pl.kernel,
        out_type=jax.ShapeDtypeStruct((2 * N, DP), jnp.float32),
        mesh=_mesh(),
        scratch_types=[
            pltpu.VMEM((CH,), jnp.int32),
            pltpu.VMEM((CH,), jnp.int32),
            pltpu.VMEM((CH, CG), jnp.float32),
            pltpu.VMEM((CH, CG), jnp.float32),
            pltpu.VMEM_SHARED((N, CG), jnp.float32),
            pltpu.SemaphoreType.DMA,
            pltpu.SemaphoreType.DMA,
            pltpu.SemaphoreType.DMA,
            pltpu.SemaphoreType.DMA,
            pltpu.SemaphoreType.DMA,
            pltpu.SemaphoreType.DMA,
        ],
    )
    def k(rows_hbm, idx_hbm, zeros_hbm, out_hbm,
          idx_a, idx_b, rows_a, rows_b, acc_sh,
          sia, sib, sla, slb, ssa, ssb):
        c = lax.axis_index("c")
        s = lax.axis_index("s")
        cb0 = c * NCHC

        def _do_group(g):
            pltpu.sync_copy(zeros_hbm, acc_sh.at[pl.ds(s * 400, 400)])

            @pl.when(s + NT < NBLK)
            def _():
                pltpu.sync_copy(zeros_hbm, acc_sh.at[pl.ds((s + NT) * 400, 400)])

            plsc.subcore_barrier()

            def loads(chunk, idx_r, rows_r, si, sl):
                off = chunk * CH
                pltpu.async_copy(idx_hbm.at[pl.ds(off, CH)], idx_r, si)
                pltpu.async_copy(
                    rows_hbm.at[pl.ds(off, CH), pl.ds(g * CG, CG)], rows_r, sl)

            def wait_loads(idx_r, rows_r, si, sl, chunk):
                off = chunk * CH
                pltpu.make_async_copy(
                    idx_hbm.at[pl.ds(off, CH)], idx_r, si).wait()
                pltpu.make_async_copy(
                    rows_hbm.at[pl.ds(off, CH), pl.ds(g * CG, CG)], rows_r, sl
                ).wait()

            def scat(idx_r, rows_r, sem):
                pltpu.async_copy(rows_r, acc_sh.at[idx_r], sem, add=True)

            def wait_scat(idx_r, rows_r, sem):
                pltpu.make_async_copy(rows_r, acc_sh.at[idx_r], sem).wait()

            loads(cb0 + s, idx_a, rows_a, sia, sla)
            loads(cb0 + s + NT, idx_b, rows_b, sib, slb)
            wait_loads(idx_a, rows_a, sia, sla, cb0 + s)
            scat(idx_a, rows_a, ssa)
            wait_loads(idx_b, rows_b, sib, slb, cb0 + s + NT)
            scat(idx_b, rows_b, ssb)

            def duo(kk, _):
                ca = cb0 + s + NT * (2 * kk)
                cb = cb0 + s + NT * (2 * kk + 1)
                wait_scat(idx_a, rows_a, ssa)
                loads(ca, idx_a, rows_a, sia, sla)
                wait_scat(idx_b, rows_b, ssb)
                loads(cb, idx_b, rows_b, sib, slb)
                wait_loads(idx_a, rows_a, sia, sla, ca)
                scat(idx_a, rows_a, ssa)
                wait_loads(idx_b, rows_b, sib, slb, cb)
                scat(idx_b, rows_b, ssb)
                return 0

            lax.fori_loop(1, FULL // 2, duo, 0)

            ct_a = cb0 + s + NT * (FULL - 1)
            wait_scat(idx_a, rows_a, ssa)
            loads(ct_a, idx_a, rows_a, sia, sla)
            wait_loads(idx_a, rows_a, sia, sla, ct_a)
            scat(idx_a, rows_a, ssa)

            @pl.when(s < EXTRA)
            def _():
                ct = cb0 + s + NT * FULL
                wait_scat(idx_b, rows_b, ssb)
                loads(ct, idx_b, rows_b, sib, slb)
                wait_loads(idx_b, rows_b, sib, slb, ct)
                scat(idx_b, rows_b, ssb)

            wait_scat(idx_a, rows_a, ssa)
            wait_scat(idx_b, rows_b, ssb)
            plsc.subcore_barrier()

            pltpu.sync_copy(
                acc_sh.at[pl.ds(s * 400, 400)],
                out_hbm.at[pl.ds(c * N + s * 400, 400), pl.ds(g * CG, CG)])

            @pl.when(s + NT < NBLK)
            def _():
                pltpu.sync_copy(
                    acc_sh.at[pl.ds((s + NT) * 400, 400)],
                    out_hbm.at[pl.ds(c * N + (s + NT) * 400, 400),
                               pl.ds(g * CG, CG)])

            plsc.subcore_barrier()

        for g in range(DP // CG):
            _do_group(g)

    zeros = jnp.zeros((400, CG), jnp.float32)
    return k(rows, idx, zeros)


def _sc_combine_gather(partials, idx):
    CH = 128
    NCHC = E2 // CH // 2
    NT = 16
    FULL = NCHC // NT
    EXTRA = NCHC - FULL * NT
    CG = 128
    NBLK = N // 400
    NCB = N // CH
    TAIL = N - NCB * CH

    @functools.partial(
        pl.kernel,
        out_type=jax.ShapeDtypeStruct((E2, DP), jnp.float32),
        mesh=_mesh(),
        scratch_types=[
            pltpu.VMEM((CH,), jnp.int32),
            pltpu.VMEM((CH,), jnp.int32),
            pltpu.VMEM((CH, CG), jnp.float32),
            pltpu.VMEM((CH, CG), jnp.float32),
            pltpu.VMEM_SHARED((N, CG), jnp.float32),
            pltpu.SemaphoreType.DMA,
            pltpu.SemaphoreType.DMA,
            pltpu.SemaphoreType.DMA,
            pltpu.SemaphoreType.DMA,
            pltpu.SemaphoreType.DMA,
            pltpu.SemaphoreType.DMA,
        ],
    )
    def k(part_hbm, idx_hbm, iota_hbm, out_hbm,
          idx_a, idx_b, rows_a, rows_b, acc_sh,
          sia, sib, sla, slb, ssa, ssb):
        c = lax.axis_index("c")
        s = lax.axis_index("s")
        cb0 = c * NCHC

        def _do_group(g):
            pltpu.sync_copy(
                part_hbm.at[pl.ds(s * 400, 400), pl.ds(g * CG, CG)],
                acc_sh.at[pl.ds(s * 400, 400)])

            @pl.when(s + NT < NBLK)
            def _():
                pltpu.sync_copy(
                    part_hbm.at[pl.ds((s + NT) * 400, 400), pl.ds(g * CG, CG)],
                    acc_sh.at[pl.ds((s + NT) * 400, 400)])

            plsc.subcore_barrier()

            def add_chunk(chk, ln):
                offs = idx_a.at[pl.ds(0, ln)]
                pltpu.sync_copy(iota_hbm.at[pl.ds(chk * CH, ln)], offs)
                pltpu.sync_copy(
                    part_hbm.at[pl.ds(N + chk * CH, ln), pl.ds(g * CG, CG)],
                    rows_a.at[pl.ds(0, ln)])
                pltpu.async_copy(
                    rows_a.at[pl.ds(0, ln)], acc_sh.at[offs], ssa, add=True)
                pltpu.make_async_copy(
                    rows_a.at[pl.ds(0, ln)], acc_sh.at[offs], ssa).wait()

            for j in range(NCB // NT):
                add_chunk(s + NT * j, CH)

            @pl.when(s < NCB - (NCB // NT) * NT)
            def _():
                add_chunk((NCB // NT) * NT + s, CH)

            @pl.when(s == NT - 1)
            def _():
                add_chunk(NCB, TAIL)

            plsc.subcore_barrier()

            def gat(chunk, idx_r, rows_r, sem):
                del chunk
                pltpu.async_copy(acc_sh.at[idx_r], rows_r, sem)

            def wait_gat(idx_r, rows_r, sem):
                pltpu.make_async_copy(acc_sh.at[idx_r], rows_r, sem).wait()

            def wr(chunk, rows_r, sem):
                pltpu.async_copy(
                    rows_r,
                    out_hbm.at[pl.ds(chunk * CH, CH), pl.ds(g * CG, CG)], sem)

            def wait_wr(chunk, rows_r, sem):
                pltpu.make_async_copy(
                    rows_r,
                    out_hbm.at[pl.ds(chunk * CH, CH), pl.ds(g * CG, CG)],
                    sem).wait()

            def aidx(chunk, idx_r, si):
                pltpu.async_copy(idx_hbm.at[pl.ds(chunk * CH, CH)], idx_r, si)

            def wait_aidx(chunk, idx_r, si):
                pltpu.make_async_copy(
                    idx_hbm.at[pl.ds(chunk * CH, CH)], idx_r, si).wait()

            aidx(cb0 + s, idx_a, sia)
            aidx(cb0 + s + NT, idx_b, sib)
            wait_aidx(cb0 + s, idx_a, sia)
            gat(cb0 + s, idx_a, rows_a, sla)
            wait_aidx(cb0 + s + NT, idx_b, sib)
            gat(cb0 + s + NT, idx_b, rows_b, slb)
            wait_gat(idx_a, rows_a, sla)
            wr(cb0 + s, rows_a, ssa)
            wait_gat(idx_b, rows_b, slb)
            wr(cb0 + s + NT, rows_b, ssb)

            def duo(kk, _):
                ca = cb0 + s + NT * (2 * kk)
                cb = cb0 + s + NT * (2 * kk + 1)
                wait_wr(ca - 2 * NT, rows_a, ssa)
                aidx(ca, idx_a, sia)
                wait_wr(cb - 2 * NT, rows_b, ssb)
                aidx(cb, idx_b, sib)
                wait_aidx(ca, idx_a, sia)
                gat(ca, idx_a, rows_a, sla)
                wait_aidx(cb, idx_b, sib)
                gat(cb, idx_b, rows_b, slb)
                wait_gat(idx_a, rows_a, sla)
                wr(ca, rows_a, ssa)
                wait_gat(idx_b, rows_b, slb)
                wr(cb, rows_b, ssb)
                return 0

            lax.fori_loop(1, FULL // 2, duo, 0)

            ct_a = cb0 + s + NT * (FULL - 1)
            wait_wr(ct_a - 2 * NT, rows_a, ssa)
            aidx(ct_a, idx_a, sia)
            wait_aidx(ct_a, idx_a, sia)
            gat(ct_a, idx_a, rows_a, sla)
            wait_gat(idx_a, rows_a, sla)
            wr(ct_a, rows_a, ssa)

            @pl.when(s < EXTRA)
            def _():
                ct = cb0 + s + NT * FULL
                wait_wr(ct - 2 * NT, rows_b, ssb)
                aidx(ct, idx_b, sib)
                wait_aidx(ct, idx_b, sib)
                gat(ct, idx_b, rows_b, slb)
                wait_gat(idx_b, rows_b, slb)
                wr(ct, rows_b, ssb)

            wait_wr(ct_a, rows_a, ssa)

            @pl.when(s < EXTRA)
            def _():
                wait_wr(cb0 + s + NT * FULL, rows_b, ssb)

            @pl.when(s >= EXTRA)
            def _():
                wait_wr(cb0 + s + NT * (FULL - 2), rows_b, ssb)

            plsc.subcore_barrier()

        for g in range(DP // CG):
            _do_group(g)

    return k(partials, idx, jnp.arange(N, dtype=jnp.int32))


def _mm_small(x, w):
    B = 1000
    K = x.shape[1]
    W = w.shape[1]

    def body(x_ref, w_ref, o_ref):
        o_ref[...] = jnp.dot(x_ref[...], w_ref[...],
                             preferred_element_type=jnp.float32)

    return pl.pallas_call(
        body,
        grid=(x.shape[0] // B,),
        in_specs=[pl.BlockSpec((B, K), lambda i: (i, 0)),
                  pl.BlockSpec((K, W), lambda i: (0, 0))],
        out_specs=pl.BlockSpec((B, W), lambda i: (i, 0)),
        out_shape=jax.ShapeDtypeStruct((x.shape[0], W), jnp.float32),
    )(x, w)


def _tc_init(pg, e2, w_ie):
    B = 1000
    DE = e2.shape[1]

    def body(pg_ref, e_ref, w_ref, h0_ref, h_ref):
        h0 = pg_ref[...] + jnp.dot(e_ref[...], w_ref[...],
                                   preferred_element_type=jnp.float32)
        h0_ref[...] = h0.astype(jnp.bfloat16)
        h_ref[...] = jnp.maximum(h0, 0.0)

    return pl.pallas_call(
        body,
        grid=(E2 // B,),
        in_specs=[pl.BlockSpec((B, DP), lambda i: (i, 0)),
                  pl.BlockSpec((B, DE), lambda i: (i, 0)),
                  pl.BlockSpec((DE, DP), lambda i: (0, 0))],
        out_specs=[pl.BlockSpec((B, DP), lambda i: (i, 0)),
                   pl.BlockSpec((B, DP), lambda i: (i, 0))],
        out_shape=[jax.ShapeDtypeStruct((E2, DP), jnp.bfloat16),
                   jax.ShapeDtypeStruct((E2, DP), jnp.float32)],
    )(pg, e2, w_ie)


def _tc_step(mg, h, h0, w_h):
    B = 1000
    NB = E2 // B

    def body(mg_ref, hsw_ref, h0_ref, w_ref, o_ref):
        x = mg_ref[...] - hsw_ref[...]
        y = jnp.dot(x, w_ref[...], preferred_element_type=jnp.float32)
        o_ref[...] = jnp.maximum(h0_ref[...].astype(jnp.float32) + y, 0.0)

    return pl.pallas_call(
        body,
        grid=(NB,),
        in_specs=[pl.BlockSpec((B, DP), lambda i: (i, 0)),
                  pl.BlockSpec((B, DP), lambda i: ((i + NB // 2) % NB, 0)),
                  pl.BlockSpec((B, DP), lambda i: (i, 0)),
                  pl.BlockSpec((DP, DP), lambda i: (0, 0))],
        out_specs=pl.BlockSpec((B, DP), lambda i: (i, 0)),
        out_shape=jax.ShapeDtypeStruct((E2, DP), jnp.float32),
    )(mg, h, h0, w_h)


def _tc_final(v, parts, w_ov, w_om, b_o):
    B = 1000
    DV = v.shape[1]
    NB = N // B

    def body(v_ref, m0_ref, m1_ref, wv_ref, wm_ref, b_ref, o_ref):
        y = jnp.dot(v_ref[...], wv_ref[...], preferred_element_type=jnp.float32)
        mv = m0_ref[...] + m1_ref[...]
        y += jnp.dot(mv, wm_ref[...], preferred_element_type=jnp.float32)
        o_ref[...] = jnp.maximum(y + b_ref[...], 0.0)

    return pl.pallas_call(
        body,
        grid=(NB,),
        in_specs=[pl.BlockSpec((B, DV), lambda i: (i, 0)),
                  pl.BlockSpec((B, DP), lambda i: (i, 0)),
                  pl.BlockSpec((B, DP), lambda i: (i + NB, 0)),
                  pl.BlockSpec((DV, DH), lambda i: (0, 0)),
                  pl.BlockSpec((DP, DH), lambda i: (0, 0)),
                  pl.BlockSpec((1, DH), lambda i: (0, 0))],
        out_specs=pl.BlockSpec((B, DH), lambda i: (i, 0)),
        out_shape=jax.ShapeDtypeStruct((N, DH), jnp.float32),
    )(v, parts, parts, w_ov, w_om, b_o)


def kernel(V, E, edge_index, rev_edge_index, W_i, W_h, W_o, b_o):
    del rev_edge_index
    DV = V.shape[1]
    src = edge_index[0]
    dst = edge_index[1]
    src2 = src.reshape(-1, 2).T.reshape(-1)
    dst2 = dst.reshape(-1, 2).T.reshape(-1)
    e2 = E.reshape(-1, 2, E.shape[1]).transpose(1, 0, 2).reshape(E2, -1)

    pad = ((0, 0), (0, DP - DH))
    w_iv = jnp.pad(W_i[:DV], pad)
    w_ie = jnp.pad(W_i[DV:], pad)
    w_h = jnp.pad(W_h, ((0, DP - DH), (0, DP - DH)))
    w_ov = W_o[:DV]
    w_om = jnp.pad(W_o[DV:], ((0, DP - DH), (0, 0)))

    p = _mm_small(V, w_iv)
    pg = _sc_gather(p, src2)
    h0, h = _tc_init(pg, e2, w_ie)

    for _ in range(2):
        part = _sc_segsum_partial(h, dst2)
        mg = _sc_combine_gather(part, src2)
        h = _tc_step(mg, h, h0, w_h)

    part = _sc_segsum_partial(h, dst2)
    return _tc_final(V, part, w_ov, w_om, b_o.reshape(1, DH))
